# baseline (speedup 1.0000x reference)
import jax
import jax.numpy as jnp
from jax import lax
from jax.experimental import pallas as pl
from jax.experimental.pallas import tpu as pltpu, tpu_sc as plsc

_B = 1024
_D = 32
_L = 16
_NC = 2
_NS = 16
_NW = _NC * _NS
_RPW = _B // _NW
_NCHUNK = _B // _L
_TCBLK = 512


def _d2_kernel(a_ref, e_ref, out_ref):
    a = a_ref[...]
    e = e_ref[...]
    g = lax.dot_general(a, e, dimension_numbers=(((1,), (1,)), ((), ())),
                        preferred_element_type=jnp.float32)
    na = jnp.sum(a * a, axis=1, keepdims=True)
    ne = jnp.sum(e * e, axis=1).reshape(1, _B)
    d2 = jnp.maximum(na + ne - 2.0 * g, 0.0)
    lo = lax.bitcast_convert_type(d2[:, :_B // 2].astype(jnp.bfloat16),
                                  jnp.uint16).astype(jnp.uint32)
    hi = lax.bitcast_convert_type(d2[:, _B // 2:].astype(jnp.bfloat16),
                                  jnp.uint16).astype(jnp.uint32)
    out_ref[...] = lax.bitcast_convert_type(lo | (hi << 16), jnp.float32)


def _newton_sqrt(x):
    i = lax.bitcast_convert_type(x, jnp.int32)
    i = jnp.int32(0x5F3759DF) - lax.shift_right_arithmetic(i, 1)
    y = lax.bitcast_convert_type(i, jnp.float32)
    for _ in range(1):
        y = y * (1.5 - 0.5 * x * y * y)
    return x * y


def _splat(ref, idx):
    return plsc.load_gather(ref, [jnp.full((_L,), idx, jnp.int32)])


_H = _RPW // 2


def _sc_body(d2_hbm, lab_hbm, out_hbm, d2_v, lab_v, out_v,
             sem_i0, sem_i1, sem_o0):
    wid = lax.axis_index("s") * _NC + lax.axis_index("c")
    base = wid * _RPW
    cp_i0 = pltpu.async_copy(d2_hbm.at[pl.ds(base, _H)],
                             d2_v.at[pl.ds(0, _H)], sem_i0)
    cp_i1 = pltpu.async_copy(d2_hbm.at[pl.ds(base + _H, _H)],
                             d2_v.at[pl.ds(_H, _H)], sem_i1)
    pltpu.sync_copy(lab_hbm, lab_v)

    def half(r0):
        @plsc.parallel_loop(r0, r0 + _H)
        def row_body(r):
            lab_r = _splat(lab_v, base + r)

            @plsc.parallel_loop(0, _NCHUNK // 2, unroll=8)
            def chunk_body(cc):
                c0 = cc * _L
                x = d2_v[r, pl.ds(c0, _L)]
                xb = plsc.bitcast(x, jnp.bfloat16)
                xlo, xhi = plsc.unpack(xb, format=plsc.PackFormat.INTERLEAVED,
                                       preferred_element_type=jnp.float32)
                for xv, col in ((xlo, c0), (xhi, c0 + _B // 2)):
                    dist = _newton_sqrt(xv + 1e-12)
                    eq = jnp.where(lab_v[pl.ds(col, _L)] == lab_r, 1.0, 0.0)
                    diff = eq - dist
                    out_v[r, pl.ds(col, _L)] = diff * diff

    cp_i0.wait()
    half(0)
    cp_o0 = pltpu.async_copy(out_v.at[pl.ds(0, _H)],
                             out_hbm.at[pl.ds(base, _H)], sem_o0)
    cp_i1.wait()
    half(_H)
    cp_o0.wait()
    pltpu.sync_copy(out_v.at[pl.ds(_H, _H)], out_hbm.at[pl.ds(base + _H, _H)])


def kernel(embeddings, labels):
    labels = labels.astype(jnp.int32)
    d2 = pl.pallas_call(
        _d2_kernel,
        grid=(_B // _TCBLK,),
        in_specs=[
            pl.BlockSpec((_TCBLK, _D), lambda i: (i, 0)),
            pl.BlockSpec((_B, _D), lambda i: (0, 0)),
        ],
        out_specs=pl.BlockSpec((_TCBLK, _B // 2), lambda i: (i, 0)),
        out_shape=jax.ShapeDtypeStruct((_B, _B // 2), jnp.float32),
    )(embeddings, embeddings)
    mesh = plsc.VectorSubcoreMesh(
        core_axis_name="c", subcore_axis_name="s",
        num_cores=_NC, num_subcores=_NS)
    out = pl.kernel(
        _sc_body,
        out_type=jax.ShapeDtypeStruct((_B, _B), jnp.float32),
        mesh=mesh,
        compiler_params=pltpu.CompilerParams(needs_layout_passes=False, skip_device_barrier=True),
        scratch_types=[
            pltpu.VMEM((_RPW, _B // 2), jnp.float32),
            pltpu.VMEM((_B,), jnp.int32),
            pltpu.VMEM((_RPW, _B), jnp.float32),
            pltpu.SemaphoreType.DMA,
            pltpu.SemaphoreType.DMA,
            pltpu.SemaphoreType.DMA,
        ],
    )(d2, labels)
    return out.reshape(-1)


# speedup vs baseline: 1.0029x; 1.0029x over previous
import jax
import jax.numpy as jnp
from jax import lax
from jax.experimental import pallas as pl
from jax.experimental.pallas import tpu as pltpu, tpu_sc as plsc

_B = 1024
_D = 32
_L = 16
_NC = 2
_NS = 16
_NW = _NC * _NS
_RPW = _B // _NW
_NCHUNK = _B // _L
_TCBLK = 512


def _d2_kernel(a_ref, e_ref, out_ref):
    a = a_ref[...]
    e = e_ref[...]
    g = lax.dot_general(a, e, dimension_numbers=(((1,), (1,)), ((), ())),
                        preferred_element_type=jnp.float32)
    na = jnp.sum(a * a, axis=1, keepdims=True)
    ne = jnp.sum(e * e, axis=1).reshape(1, _B)
    d2 = jnp.maximum(na + ne - 2.0 * g, 0.0)
    lo = lax.bitcast_convert_type(d2[:, :_B // 2].astype(jnp.bfloat16),
                                  jnp.uint16).astype(jnp.uint32)
    hi = lax.bitcast_convert_type(d2[:, _B // 2:].astype(jnp.bfloat16),
                                  jnp.uint16).astype(jnp.uint32)
    out_ref[...] = lax.bitcast_convert_type(lo | (hi << 16), jnp.float32)


def _newton_sqrt(x):
    i = lax.bitcast_convert_type(x, jnp.int32)
    i = jnp.int32(0x5F3759DF) - lax.shift_right_arithmetic(i, 1)
    y = lax.bitcast_convert_type(i, jnp.float32)
    for _ in range(1):
        y = y * (1.5 - 0.5 * x * y * y)
    return x * y


def _splat(ref, idx):
    return plsc.load_gather(ref, [jnp.full((_L,), idx, jnp.int32)])


_H = _RPW // 2


def _sc_body(d2_hbm, lab_hbm, out_hbm, d2_v, lab_v, out_v,
             sem_i0, sem_i1, sem_o0):
    wid = lax.axis_index("s") * _NC + lax.axis_index("c")
    base = wid * _RPW
    cp_i0 = pltpu.async_copy(d2_hbm.at[pl.ds(base, _H)],
                             d2_v.at[pl.ds(0, _H)], sem_i0)
    cp_i1 = pltpu.async_copy(d2_hbm.at[pl.ds(base + _H, _H)],
                             d2_v.at[pl.ds(_H, _H)], sem_i1)
    pltpu.sync_copy(lab_hbm, lab_v)

    def half(r0):
        @plsc.parallel_loop(r0, r0 + _H)
        def row_body(r):
            lab_r = _splat(lab_v, base + r)

            @plsc.parallel_loop(0, _NCHUNK // 2, unroll=8)
            def chunk_body(cc):
                c0 = cc * _L
                x = d2_v[r, pl.ds(c0, _L)]
                xb = plsc.bitcast(x, jnp.bfloat16)
                xlo, xhi = plsc.unpack(xb, format=plsc.PackFormat.INTERLEAVED,
                                       preferred_element_type=jnp.float32)
                for xv, col in ((xlo, c0), (xhi, c0 + _B // 2)):
                    dist = _newton_sqrt(xv + 1e-12)
                    eq = jnp.where(lab_v[pl.ds(col, _L)] == lab_r, 1.0, 0.0)
                    diff = eq - dist
                    out_v[r, pl.ds(col, _L)] = diff * diff

    cp_i0.wait()
    half(0)
    cp_o0 = pltpu.async_copy(out_v.at[pl.ds(0, _H)],
                             out_hbm.at[pl.ds(base, _H)], sem_o0)
    cp_i1.wait()
    half(_H)
    cp_o0.wait()
    pltpu.sync_copy(out_v.at[pl.ds(_H, _H)], out_hbm.at[pl.ds(base + _H, _H)])


def kernel(embeddings, labels):
    labels = labels.astype(jnp.int32)
    d2 = pl.pallas_call(
        _d2_kernel,
        grid=(_B // _TCBLK,),
        in_specs=[
            pl.BlockSpec((_TCBLK, _D), lambda i: (i, 0)),
            pl.BlockSpec((_B, _D), lambda i: (0, 0)),
        ],
        out_specs=pl.BlockSpec((_TCBLK, _B // 2), lambda i: (i, 0)),
        out_shape=jax.ShapeDtypeStruct((_B, _B // 2), jnp.float32),
    )(embeddings, embeddings)
    mesh = plsc.VectorSubcoreMesh(
        core_axis_name="c", subcore_axis_name="s",
        num_cores=_NC, num_subcores=_NS)
    out = pl.kernel(
        _sc_body,
        out_type=jax.ShapeDtypeStruct((_B, _B), jnp.float32),
        mesh=mesh,
        compiler_params=pltpu.CompilerParams(needs_layout_passes=False),
        scratch_types=[
            pltpu.VMEM((_RPW, _B // 2), jnp.float32),
            pltpu.VMEM((_B,), jnp.int32),
            pltpu.VMEM((_RPW, _B), jnp.float32),
            pltpu.SemaphoreType.DMA,
            pltpu.SemaphoreType.DMA,
            pltpu.SemaphoreType.DMA,
        ],
    )(d2, labels)
    return out.reshape(-1)
